# baseline (speedup 1.0000x reference)
import dataclasses
import functools

import jax
import jax.numpy as jnp
from jax import lax
from jax.experimental import pallas as pl
from jax.experimental.pallas import tpu as pltpu
from jax.experimental.pallas import tpu_sc as plsc

_V = 100000
_D = 300
_B = 1024
_G = 8
_NG = _V // _G
_NC = 2
_NS = 16
_NW = _NC * _NS
_B_PER_W = _B // _NW

_TV = 4096
_GRID = (_V + _TV - 1) // _TV


_L = 16
_NSEM = 4


def _sc_gather_groups(table, gidx):
    mesh = plsc.VectorSubcoreMesh(core_axis_name="c", subcore_axis_name="s")

    @functools.partial(
        pl.kernel,
        mesh=mesh,
        out_type=jax.ShapeDtypeStruct((_B, _G, _D), jnp.float32),
        scratch_types=[
            pltpu.VMEM((_B_PER_W,), jnp.int32),
            pltpu.VMEM((_B_PER_W, _G, _D), jnp.float32),
        ]
        + [pltpu.SemaphoreType.DMA] * _NSEM,
        compiler_params=dataclasses.replace(
            pltpu.CompilerParams(), needs_layout_passes=False
        ),
    )
    def k(table_hbm, idx_hbm, out_hbm, idx_v, rows_v, *sems):
        wid = lax.axis_index("s") * _NC + lax.axis_index("c")
        base = wid * _B_PER_W
        pltpu.sync_copy(idx_hbm.at[pl.ds(base, _B_PER_W)], idx_v)

        lane = lax.iota(jnp.int32, _L)
        for kk in range(_B_PER_W // _L):
            v = idx_v[pl.ds(kk * _L, _L)]
            for j in range(_L):
                g = lax.reduce_max(
                    jnp.where(lane == j, v, 0), axes=(0,)
                )
                row = pl.multiple_of(g * _G, _G)
                i = kk * _L + j
                pltpu.make_async_copy(
                    table_hbm.at[pl.ds(row, _G)],
                    rows_v.at[i],
                    sems[i % _NSEM],
                ).start()

        per_sem = _B_PER_W // _NSEM
        for s in range(_NSEM):
            pltpu.make_async_copy(
                out_hbm.at[pl.ds(0, per_sem)],
                rows_v.at[pl.ds(0, per_sem)],
                sems[s],
            ).wait()

        pltpu.sync_copy(rows_v, out_hbm.at[pl.ds(base, _B_PER_W)])

    return k(table, gidx)


def _select_body(r_ref, xg_ref, xs_ref):
    r = r_ref[...]
    x = xg_ref[:, 0, :] * (r == 0).astype(jnp.float32)
    for j in range(1, _G):
        x = x + xg_ref[:, j, :] * (r == j).astype(jnp.float32)
    norm = jnp.sqrt(jnp.sum(x * x, axis=1, keepdims=True))
    scale = jnp.where(norm > 1.0, 1.0 / (norm + 1e-7), 1.0)
    xs_ref[...] = (x * scale).astype(jnp.bfloat16)


def _tc_select(r, xg):
    return pl.pallas_call(
        _select_body,
        out_shape=jax.ShapeDtypeStruct((_B, _D), jnp.bfloat16),
    )(r, xg)


def _matmul_body(xs_ref, w_ref, b_ref, o_ref):
    o_ref[...] = lax.dot_general(
        xs_ref[...], w_ref[...].astype(jnp.bfloat16),
        (((1,), (1,)), ((), ())),
        preferred_element_type=jnp.float32,
    ) + b_ref[...]


def _tc_project(xs, W, b2):
    return pl.pallas_call(
        _matmul_body,
        grid=(_GRID,),
        in_specs=[
            pl.BlockSpec((_B, _D), lambda j: (0, 0)),
            pl.BlockSpec((_TV, _D), lambda j: (j, 0)),
            pl.BlockSpec((1, _TV), lambda j: (0, j)),
        ],
        out_specs=pl.BlockSpec((_B, _TV), lambda j: (0, j)),
        out_shape=jax.ShapeDtypeStruct((_B, _V), jnp.float32),
        compiler_params=pltpu.CompilerParams(
            dimension_semantics=("arbitrary",),
        ),
    )(xs, W, b2)


def kernel(inputs_, emb_table, W, b):
    idx = inputs_.astype(jnp.int32)
    xg = _sc_gather_groups(emb_table, idx // _G)
    r = (idx % _G).reshape(_B, 1)
    xs = _tc_select(r, xg)
    return _tc_project(xs, W, b.reshape(1, _V))


# speedup vs baseline: 2.4765x; 2.4765x over previous
import dataclasses
import functools

import jax
import jax.numpy as jnp
from jax import lax
from jax.experimental import pallas as pl
from jax.experimental.pallas import tpu as pltpu
from jax.experimental.pallas import tpu_sc as plsc

_V = 100000
_D = 300
_B = 1024
_G = 8
_NG = _V // _G
_NC = 2
_NS = 16
_NW = _NC * _NS
_B_PER_W = _B // _NW

_TV = 4096
_GRID = (_V + _TV - 1) // _TV


_L = 16
_NSEM = 4


def _sc_gather_groups(table, gidx):
    mesh = plsc.VectorSubcoreMesh(core_axis_name="c", subcore_axis_name="s")

    @functools.partial(
        pl.kernel,
        mesh=mesh,
        out_type=jax.ShapeDtypeStruct((_B, _G, _D), jnp.float32),
        scratch_types=[
            pltpu.VMEM((_B_PER_W,), jnp.int32),
            pltpu.VMEM((_B_PER_W, _G, _D), jnp.float32),
        ]
        + [pltpu.SemaphoreType.DMA] * _NSEM,
        compiler_params=dataclasses.replace(
            pltpu.CompilerParams(), needs_layout_passes=False
        ),
    )
    def k(table_hbm, idx_hbm, out_hbm, idx_v, rows_v, *sems):
        wid = lax.axis_index("s") * _NC + lax.axis_index("c")
        base = wid * _B_PER_W
        pltpu.sync_copy(idx_hbm.at[pl.ds(base, _B_PER_W)], idx_v)

        lane = lax.iota(jnp.int32, _L)
        for kk in range(_B_PER_W // _L):
            v = idx_v[pl.ds(kk * _L, _L)]
            for j in range(_L):
                g = lax.reduce_max(
                    jnp.where(lane == j, v, 0), axes=(0,)
                )
                row = pl.multiple_of(g * _G, _G)
                i = kk * _L + j
                pltpu.make_async_copy(
                    table_hbm.at[pl.ds(row, _G)],
                    rows_v.at[i],
                    sems[i % _NSEM],
                ).start()

        per_sem = _B_PER_W // _NSEM
        for s in range(_NSEM):
            pltpu.make_async_copy(
                out_hbm.at[pl.ds(0, per_sem)],
                rows_v.at[pl.ds(0, per_sem)],
                sems[s],
            ).wait()

        pltpu.sync_copy(rows_v, out_hbm.at[pl.ds(base, _B_PER_W)])

    return k(table, gidx)


_K = 304


def _select_body(r_ref, xg_ref, xs_ref):
    r = r_ref[...]
    x = xg_ref[:, 0, :] * (r == 0).astype(jnp.float32)
    for j in range(1, _G):
        x = x + xg_ref[:, j, :] * (r == j).astype(jnp.float32)
    norm = jnp.sqrt(jnp.sum(x * x, axis=1, keepdims=True))
    scale = jnp.where(norm > 1.0, 1.0 / (norm + 1e-7), 1.0)
    xs = x * scale
    ones = jnp.ones((_B, 1), jnp.float32)
    zero = jnp.zeros((_B, _K - _D - 1), jnp.float32)
    xs_ref[...] = jnp.concatenate([xs, ones, zero], axis=1).astype(jnp.bfloat16)


def _tc_select(r, xg):
    return pl.pallas_call(
        _select_body,
        out_shape=jax.ShapeDtypeStruct((_B, _K), jnp.bfloat16),
    )(r, xg)


def _matmul_body(wt_ref, b_ref, xs_ref, o_ref):
    a = jnp.concatenate(
        [
            wt_ref[...],
            b_ref[...],
            jnp.zeros((_K - _D - 1, o_ref.shape[0]), jnp.float32),
        ],
        axis=0,
    ).astype(jnp.bfloat16)
    o_ref[...] = lax.dot_general(
        a, xs_ref[...], (((0,), (1,)), ((), ())),
        preferred_element_type=jnp.float32,
    )


def _tc_project(WT, b2, xs):
    return pl.pallas_call(
        _matmul_body,
        grid=(_GRID,),
        in_specs=[
            pl.BlockSpec((_D, _TV), lambda j: (0, j)),
            pl.BlockSpec((1, _TV), lambda j: (0, j)),
            pl.BlockSpec((_B, _K), lambda j: (0, 0)),
        ],
        out_specs=pl.BlockSpec((_TV, _B), lambda j: (j, 0)),
        out_shape=jax.ShapeDtypeStruct((_V, _B), jnp.float32),
        compiler_params=pltpu.CompilerParams(
            dimension_semantics=("arbitrary",),
        ),
    )(WT, b2, xs)


def kernel(inputs_, emb_table, W, b):
    idx = inputs_.astype(jnp.int32)
    xg = _sc_gather_groups(emb_table, idx // _G)
    r = (idx % _G).reshape(_B, 1)
    xs = _tc_select(r, xg)
    out_t = _tc_project(W.T, b.reshape(1, _V), xs)
    return out_t.T
